# Initial kernel scaffold; baseline (speedup 1.0000x reference)
import functools

import jax
import jax.numpy as jnp
from jax import lax
from jax.experimental import pallas as pl
from jax.experimental.pallas import tpu as pltpu
from jax.experimental.pallas import tpu_sc as plsc

NUM_CLASSES = 1000
D = 256
B = 65536
MOM = 0.9

NC = 2
NS = 16
NW = NC * NS
ROWS_PER_TILE = B // NW
CHUNK = 128
NCHUNK = ROWS_PER_TILE // CHUNK
ACC_ROWS = 1024
STRIPE = ACC_ROWS // NS
CNT_W = 16


def _sc_accumulate(zeros, labels3, ff, fr, ft):
  mesh = plsc.VectorSubcoreMesh(core_axis_name="c", subcore_axis_name="s")
  pf32 = jnp.float32

  @functools.partial(
      pl.kernel,
      mesh=mesh,
      out_type=(
          jax.ShapeDtypeStruct((NC, ACC_ROWS, D), pf32),
          jax.ShapeDtypeStruct((NC, ACC_ROWS, D), pf32),
          jax.ShapeDtypeStruct((NC, ACC_ROWS, D), pf32),
          jax.ShapeDtypeStruct((NC, ACC_ROWS, CNT_W), pf32),
      ),
      scratch_types=[
          pltpu.VMEM((NCHUNK, CHUNK), jnp.int32),
          pltpu.VMEM((CHUNK, D), pf32),
          pltpu.VMEM((CHUNK, CNT_W), pf32),
          pltpu.VMEM_SHARED((ACC_ROWS, D), pf32),
          pltpu.VMEM_SHARED((ACC_ROWS, D), pf32),
          pltpu.VMEM_SHARED((ACC_ROWS, D), pf32),
          pltpu.VMEM_SHARED((ACC_ROWS, CNT_W), pf32),
      ],
  )
  def body(zeros_hbm, labels_hbm, ff_hbm, fr_hbm, ft_hbm,
           out_f, out_r, out_t, out_c,
           lab_v, fbuf, ones_v, acc_f, acc_r, acc_t, acc_c):
    cid = lax.axis_index("c")
    sid = lax.axis_index("s")
    wid = cid * NS + sid
    r0 = sid * STRIPE

    pltpu.sync_copy(zeros_hbm.at[pl.ds(r0, STRIPE)], acc_f.at[pl.ds(r0, STRIPE)])
    pltpu.sync_copy(zeros_hbm.at[pl.ds(r0, STRIPE)], acc_r.at[pl.ds(r0, STRIPE)])
    pltpu.sync_copy(zeros_hbm.at[pl.ds(r0, STRIPE)], acc_t.at[pl.ds(r0, STRIPE)])
    pltpu.sync_copy(zeros_hbm.at[pl.ds(r0, STRIPE), pl.ds(0, CNT_W)],
                    acc_c.at[pl.ds(r0, STRIPE)])

    def fill_ones(i, carry):
      ones_v[i, :] = jnp.ones((16,), pf32)
      return carry
    lax.fori_loop(0, CHUNK, fill_ones, 0)

    pltpu.sync_copy(labels_hbm.at[wid], lab_v)

    plsc.subcore_barrier()

    base = wid * ROWS_PER_TILE

    def accum(j, carry):
      idx = lab_v.at[j]
      pltpu.sync_copy(ff_hbm.at[pl.ds(base + j * CHUNK, CHUNK)], fbuf)
      pltpu.sync_copy(fbuf, acc_f.at[idx], add=True)
      pltpu.sync_copy(fr_hbm.at[pl.ds(base + j * CHUNK, CHUNK)], fbuf)
      pltpu.sync_copy(fbuf, acc_r.at[idx], add=True)
      pltpu.sync_copy(ft_hbm.at[pl.ds(base + j * CHUNK, CHUNK)], fbuf)
      pltpu.sync_copy(fbuf, acc_t.at[idx], add=True)
      pltpu.sync_copy(ones_v, acc_c.at[idx], add=True)
      return carry
    lax.fori_loop(0, NCHUNK, accum, 0)

    plsc.subcore_barrier()

    pltpu.sync_copy(acc_f.at[pl.ds(r0, STRIPE)], out_f.at[cid, pl.ds(r0, STRIPE)])
    pltpu.sync_copy(acc_r.at[pl.ds(r0, STRIPE)], out_r.at[cid, pl.ds(r0, STRIPE)])
    pltpu.sync_copy(acc_t.at[pl.ds(r0, STRIPE)], out_t.at[cid, pl.ds(r0, STRIPE)])
    pltpu.sync_copy(acc_c.at[pl.ds(r0, STRIPE)], out_c.at[cid, pl.ds(r0, STRIPE)])

  return body(zeros, labels3, ff, fr, ft)


def _tc_combine_body(pf, pr, pt, pc, prf, prr, prt, o_ref):
  cnt = pc[0] + pc[1]
  denom = jnp.maximum(cnt[:, 0:1], 1.0)
  w_new = 1.0 - MOM
  o_ref[0] = w_new * ((pf[0] + pf[1]) / denom) + MOM * prf[...]
  o_ref[1] = w_new * ((pr[0] + pr[1]) / denom) + MOM * prr[...]
  o_ref[2] = w_new * ((pt[0] + pt[1]) / denom) + MOM * prt[...]


def kernel(feat_fusion, feat_rgb, feat_t, labels, proto_fusion, proto_rgb,
           proto_t):
  labels3 = labels.astype(jnp.int32).reshape(NW, NCHUNK, CHUNK)
  zeros = jnp.zeros((ACC_ROWS, D), jnp.float32)

  pf, pr, pt, pc = _sc_accumulate(zeros, labels3, feat_fusion, feat_rgb, feat_t)

  pad = ((0, ACC_ROWS - NUM_CLASSES), (0, 0))
  prf = jnp.pad(proto_fusion, pad)
  prr = jnp.pad(proto_rgb, pad)
  prt = jnp.pad(proto_t, pad)

  out = pl.pallas_call(
      _tc_combine_body,
      out_shape=jax.ShapeDtypeStruct((3, ACC_ROWS, D), jnp.float32),
  )(pf, pr, pt, pc, prf, prr, prt)
  return out[:, :NUM_CLASSES, :]


# speedup vs baseline: 3.6374x; 3.6374x over previous
import functools

import jax
import jax.numpy as jnp
from jax import lax
from jax.experimental import pallas as pl
from jax.experimental.pallas import tpu as pltpu
from jax.experimental.pallas import tpu_sc as plsc

NUM_CLASSES = 1000
D = 256
DH = 128
B = 65536
MOM = 0.9

NC = 2
NS = 16
NW = NC * NS
ROWS_PER_TILE = B // NW
CHUNK = 128
NCHUNK = ROWS_PER_TILE // CHUNK
ACC_ROWS = 1024
STRIPE = ACC_ROWS // NS
CNT_W = 128


def _sc_accumulate(zeros, labels1, ff, fr, ft):
  mesh = plsc.VectorSubcoreMesh(core_axis_name="c", subcore_axis_name="s")
  pf32 = jnp.float32

  @functools.partial(
      pl.kernel,
      mesh=mesh,
      out_type=(
          jax.ShapeDtypeStruct((NC, ACC_ROWS, D), pf32),
          jax.ShapeDtypeStruct((NC, ACC_ROWS, D), pf32),
          jax.ShapeDtypeStruct((NC, ACC_ROWS, D), pf32),
          jax.ShapeDtypeStruct((NC, ACC_ROWS, CNT_W), pf32),
      ),
      scratch_types=[
          pltpu.VMEM((CHUNK,), jnp.int32),
          pltpu.VMEM((CHUNK, DH), pf32),
          pltpu.VMEM((CHUNK, DH), pf32),
          pltpu.VMEM((CHUNK, CNT_W), pf32),
          pltpu.VMEM_SHARED((ACC_ROWS, DH), pf32),
          pltpu.VMEM_SHARED((ACC_ROWS, DH), pf32),
          pltpu.VMEM_SHARED((ACC_ROWS, DH), pf32),
          pltpu.VMEM_SHARED((ACC_ROWS, DH), pf32),
          pltpu.VMEM_SHARED((ACC_ROWS, DH), pf32),
          pltpu.VMEM_SHARED((ACC_ROWS, DH), pf32),
          pltpu.VMEM_SHARED((ACC_ROWS, CNT_W), pf32),
          pltpu.SemaphoreType.DMA,
      ],
  )
  def body(zeros_hbm, labels_hbm, ff_hbm, fr_hbm, ft_hbm,
           out_f, out_r, out_t, out_c,
           lab_v, fb_lo, fb_hi, ones_v,
           acc_f0, acc_f1, acc_r0, acc_r1, acc_t0, acc_t1, acc_c, sem):
    cid = lax.axis_index("c")
    sid = lax.axis_index("s")
    wid = cid * NS + sid
    r0 = sid * STRIPE
    stripe = pl.ds(r0, STRIPE)

    zsrc = zeros_hbm.at[stripe, pl.ds(0, DH)]
    pltpu.sync_copy(zsrc, acc_f0.at[stripe])
    pltpu.sync_copy(zsrc, acc_f1.at[stripe])
    pltpu.sync_copy(zsrc, acc_r0.at[stripe])
    pltpu.sync_copy(zsrc, acc_r1.at[stripe])
    pltpu.sync_copy(zsrc, acc_t0.at[stripe])
    pltpu.sync_copy(zsrc, acc_t1.at[stripe])
    pltpu.sync_copy(zsrc, acc_c.at[stripe])

    one16 = jnp.ones((16,), pf32)

    def fill_ones(q, carry):
      ones_v[q // (CNT_W // 16), pl.ds((q % (CNT_W // 16)) * 16, 16)] = one16
      return carry
    lax.fori_loop(0, CHUNK * (CNT_W // 16), fill_ones, 0)

    plsc.subcore_barrier()

    base = wid * ROWS_PER_TILE

    def accum(j, carry):
      rows = pl.ds(base + j * CHUNK, CHUNK)
      pltpu.sync_copy(labels_hbm.at[rows], lab_v)
      pltpu.sync_copy(ff_hbm.at[rows, pl.ds(0, DH)], fb_lo)
      pltpu.sync_copy(ff_hbm.at[rows, pl.ds(DH, DH)], fb_hi)
      pltpu.async_copy(fb_lo, acc_f0.at[lab_v], sem, add=True).wait()
      pltpu.async_copy(fb_hi, acc_f1.at[lab_v], sem, add=True).wait()
      pltpu.sync_copy(fr_hbm.at[rows, pl.ds(0, DH)], fb_lo)
      pltpu.sync_copy(fr_hbm.at[rows, pl.ds(DH, DH)], fb_hi)
      pltpu.async_copy(fb_lo, acc_r0.at[lab_v], sem, add=True).wait()
      pltpu.async_copy(fb_hi, acc_r1.at[lab_v], sem, add=True).wait()
      pltpu.sync_copy(ft_hbm.at[rows, pl.ds(0, DH)], fb_lo)
      pltpu.sync_copy(ft_hbm.at[rows, pl.ds(DH, DH)], fb_hi)
      pltpu.async_copy(fb_lo, acc_t0.at[lab_v], sem, add=True).wait()
      pltpu.async_copy(fb_hi, acc_t1.at[lab_v], sem, add=True).wait()
      pltpu.async_copy(ones_v, acc_c.at[lab_v], sem, add=True).wait()
      return carry
    lax.fori_loop(0, NCHUNK, accum, 0)

    plsc.subcore_barrier()

    pltpu.sync_copy(acc_f0.at[stripe], out_f.at[cid, stripe, pl.ds(0, DH)])
    pltpu.sync_copy(acc_f1.at[stripe], out_f.at[cid, stripe, pl.ds(DH, DH)])
    pltpu.sync_copy(acc_r0.at[stripe], out_r.at[cid, stripe, pl.ds(0, DH)])
    pltpu.sync_copy(acc_r1.at[stripe], out_r.at[cid, stripe, pl.ds(DH, DH)])
    pltpu.sync_copy(acc_t0.at[stripe], out_t.at[cid, stripe, pl.ds(0, DH)])
    pltpu.sync_copy(acc_t1.at[stripe], out_t.at[cid, stripe, pl.ds(DH, DH)])
    pltpu.sync_copy(acc_c.at[stripe], out_c.at[cid, stripe])

  return body(zeros, labels1, ff, fr, ft)


def _tc_combine_body(pf, pr, pt, pc, prf, prr, prt, o_ref):
  cnt = pc[0] + pc[1]
  denom = jnp.maximum(cnt[:, 0:1], 1.0)
  w_new = 1.0 - MOM
  o_ref[0] = w_new * ((pf[0] + pf[1]) / denom) + MOM * prf[...]
  o_ref[1] = w_new * ((pr[0] + pr[1]) / denom) + MOM * prr[...]
  o_ref[2] = w_new * ((pt[0] + pt[1]) / denom) + MOM * prt[...]


def kernel(feat_fusion, feat_rgb, feat_t, labels, proto_fusion, proto_rgb,
           proto_t):
  labels1 = labels.astype(jnp.int32)
  zeros = jnp.zeros((ACC_ROWS, DH), jnp.float32)

  pf, pr, pt, pc = _sc_accumulate(zeros, labels1, feat_fusion, feat_rgb,
                                  feat_t)

  pad = ((0, ACC_ROWS - NUM_CLASSES), (0, 0))
  prf = jnp.pad(proto_fusion, pad)
  prr = jnp.pad(proto_rgb, pad)
  prt = jnp.pad(proto_t, pad)

  out = pl.pallas_call(
      _tc_combine_body,
      out_shape=jax.ShapeDtypeStruct((3, ACC_ROWS, D), jnp.float32),
  )(pf, pr, pt, pc, prf, prr, prt)
  return out[:, :NUM_CLASSES, :]


# speedup vs baseline: 5.1510x; 1.4161x over previous
import functools

import jax
import jax.numpy as jnp
from jax import lax
from jax.experimental import pallas as pl
from jax.experimental.pallas import tpu as pltpu
from jax.experimental.pallas import tpu_sc as plsc

NUM_CLASSES = 1000
D = 256
DH = 128
B = 65536
MOM = 0.9

NC = 2
NS = 16
NW = NC * NS
ROWS_PER_TILE = B // NW
CHUNK = 64
NCHUNK = ROWS_PER_TILE // CHUNK
ACC_ROWS = 1024
STRIPE = ACC_ROWS // NS
CNT_W = 128

UNITS = ((0, 0), (0, 1), (0, 2), (1, 0), (1, 1), (1, 2))


def _sc_accumulate(zeros, labels1, ff, fr, ft):
  mesh = plsc.VectorSubcoreMesh(core_axis_name="c", subcore_axis_name="s")
  pf32 = jnp.float32

  @functools.partial(
      pl.kernel,
      mesh=mesh,
      out_type=(
          jax.ShapeDtypeStruct((NC, ACC_ROWS, D), pf32),
          jax.ShapeDtypeStruct((NC, ACC_ROWS, D), pf32),
          jax.ShapeDtypeStruct((NC, ACC_ROWS, D), pf32),
          jax.ShapeDtypeStruct((NC, ACC_ROWS, CNT_W), pf32),
      ),
      scratch_types=[
          pltpu.VMEM((CHUNK,), jnp.int32),
          pltpu.VMEM((CHUNK,), jnp.int32),
          pltpu.VMEM((CHUNK, DH), pf32),
          pltpu.VMEM((CHUNK, DH), pf32),
          pltpu.VMEM((CHUNK, DH), pf32),
          pltpu.VMEM((CHUNK, DH), pf32),
          pltpu.VMEM((CHUNK, CNT_W), pf32),
          pltpu.VMEM_SHARED((ACC_ROWS, DH), pf32),
          pltpu.VMEM_SHARED((ACC_ROWS, DH), pf32),
          pltpu.VMEM_SHARED((ACC_ROWS, DH), pf32),
          pltpu.VMEM_SHARED((ACC_ROWS, DH), pf32),
          pltpu.VMEM_SHARED((ACC_ROWS, DH), pf32),
          pltpu.VMEM_SHARED((ACC_ROWS, DH), pf32),
          pltpu.VMEM_SHARED((ACC_ROWS, CNT_W), pf32),
          pltpu.SemaphoreType.DMA,
          pltpu.SemaphoreType.DMA,
      ],
  )
  def body(zeros_hbm, labels_hbm, ff_hbm, fr_hbm, ft_hbm,
           out_f, out_r, out_t, out_c,
           lab0, lab1, lo0, hi0, lo1, hi1, ones_v,
           acc_f0, acc_f1, acc_r0, acc_r1, acc_t0, acc_t1, acc_c,
           gsem, ssem):
    cid = lax.axis_index("c")
    sid = lax.axis_index("s")
    wid = cid * NS + sid
    r0 = sid * STRIPE
    stripe = pl.ds(r0, STRIPE)
    base = wid * ROWS_PER_TILE

    labs = (lab0, lab1)
    bl = (lo0, lo1)
    bh = (hi0, hi1)
    streams = ((ff_hbm, acc_f0, acc_f1),
               (fr_hbm, acc_r0, acc_r1),
               (ft_hbm, acc_t0, acc_t1))

    zsrc = zeros_hbm.at[stripe, pl.ds(0, DH)]
    pltpu.sync_copy(zsrc, acc_f0.at[stripe])
    pltpu.sync_copy(zsrc, acc_f1.at[stripe])
    pltpu.sync_copy(zsrc, acc_r0.at[stripe])
    pltpu.sync_copy(zsrc, acc_r1.at[stripe])
    pltpu.sync_copy(zsrc, acc_t0.at[stripe])
    pltpu.sync_copy(zsrc, acc_t1.at[stripe])
    pltpu.sync_copy(zsrc, acc_c.at[stripe])

    one16 = jnp.ones((16,), pf32)

    def fill_ones(q, carry):
      ones_v[q // (CNT_W // 16), pl.ds((q % (CNT_W // 16)) * 16, 16)] = one16
      return carry
    lax.fori_loop(0, CHUNK * (CNT_W // 16), fill_ones, 0)

    plsc.subcore_barrier()

    def rows_of(j):
      return pl.ds(base + j * CHUNK, CHUNK)

    def issue_gather(j, s, p, dj):
      src = streams[s][0]
      if s == 0:
        pltpu.async_copy(labels_hbm.at[rows_of(j)], labs[dj], gsem)
      pltpu.async_copy(src.at[rows_of(j), pl.ds(0, DH)], bl[p], gsem)
      pltpu.async_copy(src.at[rows_of(j), pl.ds(DH, DH)], bh[p], gsem)

    def wait_gather(s, p, dj):
      src = streams[s][0]
      if s == 0:
        pltpu.make_async_copy(labels_hbm.at[pl.ds(0, CHUNK)], labs[dj], gsem).wait()
      pltpu.make_async_copy(src.at[pl.ds(0, CHUNK), pl.ds(0, DH)], bl[p], gsem).wait()
      pltpu.make_async_copy(src.at[pl.ds(0, CHUNK), pl.ds(DH, DH)], bh[p], gsem).wait()

    def issue_scatter(s, p, dj):
      _, a0, a1 = streams[s]
      pltpu.async_copy(bl[p], a0.at[labs[dj]], ssem, add=True)
      pltpu.async_copy(bh[p], a1.at[labs[dj]], ssem, add=True)
      if s == 2:
        pltpu.async_copy(ones_v, acc_c.at[labs[dj]], ssem, add=True)

    def wait_scatter(s, p, dj):
      _, a0, a1 = streams[s]
      pltpu.make_async_copy(bl[p], a0.at[labs[dj]], ssem).wait()
      pltpu.make_async_copy(bh[p], a1.at[labs[dj]], ssem).wait()
      if s == 2:
        pltpu.make_async_copy(ones_v, acc_c.at[labs[dj]], ssem).wait()

    def step(k, j0, first_pair):
      dj, s = UNITS[k]
      p = k % 2
      wait_gather(s, p, dj)
      issue_scatter(s, p, dj)
      if k == 0:
        if not first_pair:
          wait_scatter(2, 1, 1)
      else:
        pdj, ps = UNITS[k - 1]
        wait_scatter(ps, (k - 1) % 2, pdj)
      if k < 5:
        ndj, ns = UNITS[k + 1]
        issue_gather(j0 + ndj, ns, (k + 1) % 2, ndj)
      else:
        @pl.when(j0 + 2 < NCHUNK)
        def _():
          issue_gather(j0 + 2, 0, 0, 0)

    issue_gather(0, 0, 0, 0)
    for k in range(6):
      step(k, 0, first_pair=True)

    @pl.loop(2, NCHUNK, step=2)
    def _pair(j0):
      for k in range(6):
        step(k, j0, first_pair=False)

    wait_scatter(2, 1, 1)

    plsc.subcore_barrier()

    pltpu.sync_copy(acc_f0.at[stripe], out_f.at[cid, stripe, pl.ds(0, DH)])
    pltpu.sync_copy(acc_f1.at[stripe], out_f.at[cid, stripe, pl.ds(DH, DH)])
    pltpu.sync_copy(acc_r0.at[stripe], out_r.at[cid, stripe, pl.ds(0, DH)])
    pltpu.sync_copy(acc_r1.at[stripe], out_r.at[cid, stripe, pl.ds(DH, DH)])
    pltpu.sync_copy(acc_t0.at[stripe], out_t.at[cid, stripe, pl.ds(0, DH)])
    pltpu.sync_copy(acc_t1.at[stripe], out_t.at[cid, stripe, pl.ds(DH, DH)])
    pltpu.sync_copy(acc_c.at[stripe], out_c.at[cid, stripe])

  return body(zeros, labels1, ff, fr, ft)


def _tc_combine_body(pf, pr, pt, pc, prf, prr, prt, o_ref):
  cnt = pc[0] + pc[1]
  denom = jnp.maximum(cnt[:, 0:1], 1.0)
  w_new = 1.0 - MOM
  o_ref[0] = w_new * ((pf[0] + pf[1]) / denom) + MOM * prf[...]
  o_ref[1] = w_new * ((pr[0] + pr[1]) / denom) + MOM * prr[...]
  o_ref[2] = w_new * ((pt[0] + pt[1]) / denom) + MOM * prt[...]


def kernel(feat_fusion, feat_rgb, feat_t, labels, proto_fusion, proto_rgb,
           proto_t):
  labels1 = labels.astype(jnp.int32)
  zeros = jnp.zeros((ACC_ROWS, DH), jnp.float32)

  pf, pr, pt, pc = _sc_accumulate(zeros, labels1, feat_fusion, feat_rgb,
                                  feat_t)

  pad = ((0, ACC_ROWS - NUM_CLASSES), (0, 0))
  prf = jnp.pad(proto_fusion, pad)
  prr = jnp.pad(proto_rgb, pad)
  prt = jnp.pad(proto_t, pad)

  out = pl.pallas_call(
      _tc_combine_body,
      out_shape=jax.ShapeDtypeStruct((3, ACC_ROWS, D), jnp.float32),
  )(pf, pr, pt, pc, prf, prr, prt)
  return out[:, :NUM_CLASSES, :]


# speedup vs baseline: 5.2589x; 1.0209x over previous
import functools

import jax
import jax.numpy as jnp
from jax import lax
from jax.experimental import pallas as pl
from jax.experimental.pallas import tpu as pltpu
from jax.experimental.pallas import tpu_sc as plsc

NUM_CLASSES = 1000
D = 256
DH = 128
B = 65536
MOM = 0.9

NC = 2
NS = 16
NW = NC * NS
ROWS_PER_TILE = B // NW
CHUNK = 64
NCHUNK = ROWS_PER_TILE // CHUNK
ACC_ROWS = NUM_CLASSES
STRIPE = 64
LAST_STRIPE = ACC_ROWS - 15 * STRIPE
CNT_W = 128

UNITS = ((0, 0), (0, 1), (0, 2), (1, 0), (1, 1), (1, 2))


def _sc_accumulate(zeros, labels1, ff, fr, ft):
  mesh = plsc.VectorSubcoreMesh(core_axis_name="c", subcore_axis_name="s")
  pf32 = jnp.float32

  @functools.partial(
      pl.kernel,
      mesh=mesh,
      out_type=(
          jax.ShapeDtypeStruct((NC, ACC_ROWS, D), pf32),
          jax.ShapeDtypeStruct((NC, ACC_ROWS, D), pf32),
          jax.ShapeDtypeStruct((NC, ACC_ROWS, D), pf32),
          jax.ShapeDtypeStruct((NC, ACC_ROWS, CNT_W), pf32),
      ),
      scratch_types=[
          pltpu.VMEM((CHUNK,), jnp.int32),
          pltpu.VMEM((CHUNK,), jnp.int32),
          pltpu.VMEM((CHUNK, DH), pf32),
          pltpu.VMEM((CHUNK, DH), pf32),
          pltpu.VMEM((CHUNK, DH), pf32),
          pltpu.VMEM((CHUNK, DH), pf32),
          pltpu.VMEM((CHUNK, CNT_W), pf32),
          pltpu.VMEM_SHARED((ACC_ROWS, DH), pf32),
          pltpu.VMEM_SHARED((ACC_ROWS, DH), pf32),
          pltpu.VMEM_SHARED((ACC_ROWS, DH), pf32),
          pltpu.VMEM_SHARED((ACC_ROWS, DH), pf32),
          pltpu.VMEM_SHARED((ACC_ROWS, DH), pf32),
          pltpu.VMEM_SHARED((ACC_ROWS, DH), pf32),
          pltpu.VMEM_SHARED((ACC_ROWS, CNT_W), pf32),
          pltpu.SemaphoreType.DMA,
          pltpu.SemaphoreType.DMA,
      ],
  )
  def body(zeros_hbm, labels_hbm, ff_hbm, fr_hbm, ft_hbm,
           out_f, out_r, out_t, out_c,
           lab0, lab1, lo0, hi0, lo1, hi1, ones_v,
           acc_f0, acc_f1, acc_r0, acc_r1, acc_t0, acc_t1, acc_c,
           gsem, ssem):
    cid = lax.axis_index("c")
    sid = lax.axis_index("s")
    wid = cid * NS + sid
    r0 = sid * STRIPE
    base = wid * ROWS_PER_TILE

    labs = (lab0, lab1)
    bl = (lo0, lo1)
    bh = (hi0, hi1)
    streams = ((ff_hbm, acc_f0, acc_f1),
               (fr_hbm, acc_r0, acc_r1),
               (ft_hbm, acc_t0, acc_t1))

    def zero_all(rows):
      zsrc = zeros_hbm.at[rows, pl.ds(0, DH)]
      pltpu.sync_copy(zsrc, acc_f0.at[rows])
      pltpu.sync_copy(zsrc, acc_f1.at[rows])
      pltpu.sync_copy(zsrc, acc_r0.at[rows])
      pltpu.sync_copy(zsrc, acc_r1.at[rows])
      pltpu.sync_copy(zsrc, acc_t0.at[rows])
      pltpu.sync_copy(zsrc, acc_t1.at[rows])
      pltpu.sync_copy(zsrc, acc_c.at[rows])

    @pl.when(sid < NS - 1)
    def _():
      zero_all(pl.ds(r0, STRIPE))

    @pl.when(sid == NS - 1)
    def _():
      zero_all(pl.ds((NS - 1) * STRIPE, LAST_STRIPE))

    one16 = jnp.ones((16,), pf32)

    def fill_ones(q, carry):
      ones_v[q // (CNT_W // 16), pl.ds((q % (CNT_W // 16)) * 16, 16)] = one16
      return carry
    lax.fori_loop(0, CHUNK * (CNT_W // 16), fill_ones, 0)

    plsc.subcore_barrier()

    def rows_of(j):
      return pl.ds(base + j * CHUNK, CHUNK)

    def issue_gather(j, s, p, dj):
      src = streams[s][0]
      if s == 0:
        pltpu.async_copy(labels_hbm.at[rows_of(j)], labs[dj], gsem)
      pltpu.async_copy(src.at[rows_of(j), pl.ds(0, DH)], bl[p], gsem)
      pltpu.async_copy(src.at[rows_of(j), pl.ds(DH, DH)], bh[p], gsem)

    def wait_gather(s, p, dj):
      src = streams[s][0]
      if s == 0:
        pltpu.make_async_copy(labels_hbm.at[pl.ds(0, CHUNK)], labs[dj], gsem).wait()
      pltpu.make_async_copy(src.at[pl.ds(0, CHUNK), pl.ds(0, DH)], bl[p], gsem).wait()
      pltpu.make_async_copy(src.at[pl.ds(0, CHUNK), pl.ds(DH, DH)], bh[p], gsem).wait()

    def issue_scatter(s, p, dj):
      _, a0, a1 = streams[s]
      pltpu.async_copy(bl[p], a0.at[labs[dj]], ssem, add=True)
      pltpu.async_copy(bh[p], a1.at[labs[dj]], ssem, add=True)
      if s == 2:
        pltpu.async_copy(ones_v, acc_c.at[labs[dj]], ssem, add=True)

    def wait_scatter(s, p, dj):
      _, a0, a1 = streams[s]
      pltpu.make_async_copy(bl[p], a0.at[labs[dj]], ssem).wait()
      pltpu.make_async_copy(bh[p], a1.at[labs[dj]], ssem).wait()
      if s == 2:
        pltpu.make_async_copy(ones_v, acc_c.at[labs[dj]], ssem).wait()

    def step(k, j0, first_pair):
      dj, s = UNITS[k]
      p = k % 2
      wait_gather(s, p, dj)
      issue_scatter(s, p, dj)
      if k == 0:
        if not first_pair:
          wait_scatter(2, 1, 1)
      else:
        pdj, ps = UNITS[k - 1]
        wait_scatter(ps, (k - 1) % 2, pdj)
      if k < 5:
        ndj, ns = UNITS[k + 1]
        issue_gather(j0 + ndj, ns, (k + 1) % 2, ndj)
      else:
        @pl.when(j0 + 2 < NCHUNK)
        def _():
          issue_gather(j0 + 2, 0, 0, 0)

    issue_gather(0, 0, 0, 0)
    for k in range(6):
      step(k, 0, first_pair=True)

    @pl.loop(2, NCHUNK, step=2)
    def _pair(j0):
      for k in range(6):
        step(k, j0, first_pair=False)

    wait_scatter(2, 1, 1)

    plsc.subcore_barrier()

    def write_all(rows):
      pltpu.sync_copy(acc_f0.at[rows], out_f.at[cid, rows, pl.ds(0, DH)])
      pltpu.sync_copy(acc_f1.at[rows], out_f.at[cid, rows, pl.ds(DH, DH)])
      pltpu.sync_copy(acc_r0.at[rows], out_r.at[cid, rows, pl.ds(0, DH)])
      pltpu.sync_copy(acc_r1.at[rows], out_r.at[cid, rows, pl.ds(DH, DH)])
      pltpu.sync_copy(acc_t0.at[rows], out_t.at[cid, rows, pl.ds(0, DH)])
      pltpu.sync_copy(acc_t1.at[rows], out_t.at[cid, rows, pl.ds(DH, DH)])
      pltpu.sync_copy(acc_c.at[rows], out_c.at[cid, rows])

    @pl.when(sid < NS - 1)
    def _():
      write_all(pl.ds(r0, STRIPE))

    @pl.when(sid == NS - 1)
    def _():
      write_all(pl.ds((NS - 1) * STRIPE, LAST_STRIPE))

  return body(zeros, labels1, ff, fr, ft)


def _tc_combine_body(pf, pr, pt, pc, prf, prr, prt, o_ref):
  cnt = pc[0] + pc[1]
  denom = jnp.maximum(cnt[:, 0:1], 1.0)
  w_new = 1.0 - MOM
  o_ref[0] = w_new * ((pf[0] + pf[1]) / denom) + MOM * prf[...]
  o_ref[1] = w_new * ((pr[0] + pr[1]) / denom) + MOM * prr[...]
  o_ref[2] = w_new * ((pt[0] + pt[1]) / denom) + MOM * prt[...]


def kernel(feat_fusion, feat_rgb, feat_t, labels, proto_fusion, proto_rgb,
           proto_t):
  labels1 = labels.astype(jnp.int32)
  zeros = jnp.zeros((STRIPE * NS, DH), jnp.float32)

  pf, pr, pt, pc = _sc_accumulate(zeros, labels1, feat_fusion, feat_rgb,
                                  feat_t)

  out = pl.pallas_call(
      _tc_combine_body,
      out_shape=jax.ShapeDtypeStruct((3, ACC_ROWS, D), jnp.float32),
  )(pf, pr, pt, pc, proto_fusion, proto_rgb, proto_t)
  return out
